# baseline (speedup 1.0000x reference)
import functools

import jax
import jax.numpy as jnp
from jax import lax
from jax.experimental import pallas as pl
from jax.experimental.pallas import tpu as pltpu
from jax.experimental.pallas import tpu_sc as plsc

BATCH = 16384
DIM = 32

_INFO = plsc.get_sparse_core_info()
_NC = _INFO.num_cores
_NS = _INFO.num_subcores
_NW = _NC * _NS
_BPW = BATCH // _NW
_CHUNK = 128
_NCHUNK = _BPW // _CHUNK


def _gather3_body(u_idx, p_idx, n_idx, uT, iT,
                  u_out, p_out, n_out,
                  ix_u, ix_p, ix_n, rows_u, rows_p, rows_n,
                  sem_u, sem_p, sem_n, sem_s):
    wid = lax.axis_index("s") * _NC + lax.axis_index("c")
    cbase = wid * _NCHUNK

    pltpu.sync_copy(u_idx.at[pl.ds(cbase, _NCHUNK)], ix_u)
    pltpu.sync_copy(p_idx.at[pl.ds(cbase, _NCHUNK)], ix_p)
    pltpu.sync_copy(n_idx.at[pl.ds(cbase, _NCHUNK)], ix_n)

    for idx2, tbl, rows, sem in ((ix_u, uT, rows_u, sem_u),
                                 (ix_p, iT, rows_p, sem_p),
                                 (ix_n, iT, rows_n, sem_n)):
        def issue(d, carry, idx2=idx2, tbl=tbl, rows=rows, sem=sem):
            k = d // 8
            s = d % 8
            for jc in range(_NCHUNK):
                pltpu.async_copy(
                    tbl.at[d].at[idx2.at[jc]],
                    rows.at[k, s, pl.ds(jc * _CHUNK, _CHUNK)], sem)
            return carry
        lax.fori_loop(0, DIM, issue, 0)

    stores = []
    for tbl, rows, out, sem in ((uT, rows_u, u_out, sem_u),
                                (iT, rows_p, p_out, sem_p),
                                (iT, rows_n, n_out, sem_n)):
        pltpu.make_async_copy(tbl.at[0].at[pl.ds(0, 4 * 8 * _BPW)], rows,
                              sem).wait()
        for k in range(4):
            for c4 in range(_NCHUNK):
                stores.append(pltpu.async_copy(
                    rows.at[k, :, pl.ds(c4 * _CHUNK, _CHUNK)],
                    out.at[k, _NCHUNK * wid + c4], sem_s))
    for st in stores:
        st.wait()


@jax.jit
def _gather3(u_idx, p_idx, n_idx, uT, iT):
    out_ty = jax.ShapeDtypeStruct((4, BATCH // 128, 8, 128), jnp.float32)
    run = pl.kernel(
        _gather3_body,
        mesh=plsc.VectorSubcoreMesh(core_axis_name="c", subcore_axis_name="s"),
        compiler_params=pltpu.CompilerParams(use_tc_tiling_on_sc=False),
        out_type=(out_ty, out_ty, out_ty),
        scratch_types=[
            pltpu.VMEM((_NCHUNK, _CHUNK), jnp.int32),
            pltpu.VMEM((_NCHUNK, _CHUNK), jnp.int32),
            pltpu.VMEM((_NCHUNK, _CHUNK), jnp.int32),
            pltpu.VMEM((4, 8, _BPW), jnp.float32),
            pltpu.VMEM((4, 8, _BPW), jnp.float32),
            pltpu.VMEM((4, 8, _BPW), jnp.float32),
            pltpu.SemaphoreType.DMA,
            pltpu.SemaphoreType.DMA,
            pltpu.SemaphoreType.DMA,
            pltpu.SemaphoreType.DMA,
        ],
    )
    return run(u_idx, p_idx, n_idx, uT, iT)


def kernel(users, pos_items, neg_items, user_weight, item_weight):
    u = users.astype(jnp.int32).reshape(_NW * _NCHUNK, _CHUNK)
    p = pos_items.astype(jnp.int32).reshape(_NW * _NCHUNK, _CHUNK)
    n = neg_items.astype(jnp.int32).reshape(_NW * _NCHUNK, _CHUNK)
    u4, p4, n4 = _gather3(u, p, n, user_weight.T, item_weight.T)

    def unpack(x4):
        return x4.transpose(1, 3, 0, 2).reshape(BATCH, DIM)

    return (unpack(u4), unpack(p4), unpack(n4))
